# baseline (speedup 1.0000x reference)
import functools

import jax
import jax.numpy as jnp
import numpy as np
from jax.experimental import pallas as pl
from jax.experimental.pallas import tpu as pltpu

_CD = jnp.bfloat16
_NT = 3
_EPS = 1e-5


def _compiler_params():
    return pltpu.CompilerParams(
        dimension_semantics=("parallel",),
        vmem_limit_bytes=56 * 1024 * 1024)


def _up_vec(n, m):
    k = np.arange(m) // 2
    p = np.arange(m) % 2
    nb = np.where(p == 0, np.maximum(k - 1, 0), np.minimum(k + 1, n - 1))
    r = np.arange(n)[:, None]
    return 0.75 * (r == k[None, :]) + 0.25 * (r == nb[None, :])


def _upsample_matrix(h, w, wp):
    uv = _up_vec(h, 2 * h)
    uh = np.pad(_up_vec(w, 2 * w), ((0, 0), (1, 1)))
    b = uv[:, None, :, None] * uh[None, :, None, :]
    return jnp.asarray(b.reshape(h * w, 2 * h * wp), dtype=_CD)


def _fold_bn_in_kernel(psin_ref, pssin_ref, gamma_ref, beta_ref, count):
    s = jnp.sum(psin_ref[...], axis=0)
    ss = jnp.sum(pssin_ref[...], axis=0)
    mean = s / count
    var = jnp.maximum(ss / count - mean * mean, 0.0)
    scale = gamma_ref[...] / jnp.sqrt(var + _EPS)
    shift = beta_ref[...] - mean * scale
    return scale, shift


def _make_xfull(a, *, wp):
    cin, lf = a.shape
    return jnp.concatenate(
        [jnp.zeros((cin, wp + 1), _CD), a,
         jnp.zeros((cin, wp + 1), _CD)], axis=1)


def _conv_tiles(xfull, w_ref, y_ref, ps_ref, pss_ref, *, wp, lt):
    cin = xfull.shape[0]
    ps = jnp.zeros((1, y_ref.shape[1]), jnp.float32)
    pss = jnp.zeros((1, y_ref.shape[1]), jnp.float32)
    for ti in range(_NT):
        f0 = ti * lt
        stk = jnp.concatenate(
            [xfull[:, f0 + (k // 3) * wp + (k % 3):
                   f0 + (k // 3) * wp + (k % 3) + lt]
             for k in range(9)], axis=0)
        acc = jnp.dot(w_ref[...], stk, preferred_element_type=jnp.float32)
        y_ref[0, :, f0:f0 + lt] = acc.astype(y_ref.dtype)
        col = (jax.lax.broadcasted_iota(jnp.int32, (1, lt), 1) + f0) % wp
        am = jnp.where((col >= 1) & (col <= wp - 2), acc, 0.0)
        ps = ps + jnp.sum(am, axis=1, keepdims=True).reshape(1, -1)
        pss = pss + jnp.sum(am * am, axis=1, keepdims=True).reshape(1, -1)
    ps_ref[0] = ps
    pss_ref[0] = pss


def _conv1_kernel(x_ref, prev_ref, b_ref, w_ref, y_ref, ps_ref, pss_ref,
                  *, wp, lt):
    ax = jnp.dot(x_ref[0].astype(_CD), b_ref[...],
                 preferred_element_type=jnp.float32).astype(_CD)
    a = jnp.concatenate([ax, prev_ref[0]], axis=0)
    _conv_tiles(_make_xfull(a, wp=wp), w_ref, y_ref, ps_ref, pss_ref,
                wp=wp, lt=lt)


def _conv_mid_kernel(yin_ref, w_ref, psin_ref, pssin_ref, gamma_ref, beta_ref,
                     y_ref, ps_ref, pss_ref, *, wp, lt, count):
    lf = yin_ref.shape[2]
    scale, shift = _fold_bn_in_kernel(psin_ref, pssin_ref, gamma_ref,
                                      beta_ref, count)
    v = yin_ref[0].astype(jnp.float32) * scale.reshape(-1, 1) + shift.reshape(-1, 1)
    v = jnp.maximum(v, 0.0)
    col = jax.lax.broadcasted_iota(jnp.int32, (1, lf), 1) % wp
    a = jnp.where((col >= 1) & (col <= wp - 2), v, 0.0).astype(_CD)
    _conv_tiles(_make_xfull(a, wp=wp), w_ref, y_ref, ps_ref, pss_ref,
                wp=wp, lt=lt)


def _epilogue_kernel(yin_ref, psin_ref, pssin_ref, gamma_ref, beta_ref,
                     o_ref, *, wp, count):
    c, h2, w2 = o_ref.shape[1], o_ref.shape[2], o_ref.shape[3]
    scale, shift = _fold_bn_in_kernel(psin_ref, pssin_ref, gamma_ref,
                                      beta_ref, count)
    v = jnp.maximum(yin_ref[0].astype(jnp.float32) * scale.reshape(-1, 1)
                    + shift.reshape(-1, 1), 0.0)
    o_ref[0] = v.reshape(c, h2, wp)[:, :, 1:w2 + 1]


def _stack_taps(w_oihw):
    cout, cin = w_oihw.shape[0], w_oihw.shape[1]
    return jnp.transpose(w_oihw, (0, 2, 3, 1)).reshape(cout, 9 * cin).astype(_CD)


def kernel(prev, x, w1, b1, gamma1, beta1, w2, b2, gamma2, beta2,
           w3, b3, gamma3, beta3):
    n, cx, h, w = x.shape
    cp = prev.shape[1]
    h2, w2s = 2 * h, 2 * w
    wp = w2s + 2
    lf = h2 * wp
    lt = lf // _NT
    count = n * h2 * w2s
    cin1 = cx + cp
    cout = w1.shape[0]

    xf = x.reshape(n, cx, h * w)
    prevf = jnp.pad(prev, ((0, 0), (0, 0), (0, 0), (1, 1))
                    ).reshape(n, cp, lf).astype(_CD)
    bmat = _upsample_matrix(h, w, wp)
    wall = jnp.concatenate(
        [_stack_taps(w1), _stack_taps(w2), _stack_taps(w3)], axis=1)

    def conv_out_shape():
        return (jax.ShapeDtypeStruct((n, cout, lf), _CD),
                jax.ShapeDtypeStruct((n, 1, cout), jnp.float32),
                jax.ShapeDtypeStruct((n, 1, cout), jnp.float32))

    def conv_out_specs():
        return (pl.BlockSpec((1, cout, lf), lambda i: (i, 0, 0)),
                pl.BlockSpec((1, 1, cout), lambda i: (i, 0, 0)),
                pl.BlockSpec((1, 1, cout), lambda i: (i, 0, 0)))

    y1, ps, pss = pl.pallas_call(
        functools.partial(_conv1_kernel, wp=wp, lt=lt),
        out_shape=conv_out_shape(),
        grid=(n,),
        in_specs=[
            pl.BlockSpec((1, cx, h * w), lambda i: (i, 0, 0)),
            pl.BlockSpec((1, cp, lf), lambda i: (i, 0, 0)),
            pl.BlockSpec((h * w, lf), lambda i: (0, 0)),
            pl.BlockSpec((cout, 9 * cin1), lambda i: (0, 0)),
        ],
        out_specs=conv_out_specs(),
        compiler_params=_compiler_params(),
    )(xf, prevf, bmat, wall)

    yk = y1
    for li, (wi, gi, bi) in enumerate(((w2, gamma1, beta1),
                                       (w3, gamma2, beta2))):
        ci = wi.shape[1]
        wblk = (9 * cin1) // (9 * ci) + li
        yk, ps, pss = pl.pallas_call(
            functools.partial(_conv_mid_kernel, wp=wp, lt=lt, count=count),
            out_shape=conv_out_shape(),
            grid=(n,),
            in_specs=[
                pl.BlockSpec((1, ci, lf), lambda i: (i, 0, 0)),
                pl.BlockSpec((cout, 9 * ci),
                             functools.partial(lambda b, i: (0, b), wblk)),
                pl.BlockSpec((n, 1, ci), lambda i: (0, 0, 0)),
                pl.BlockSpec((n, 1, ci), lambda i: (0, 0, 0)),
                pl.BlockSpec((1, ci), lambda i: (0, 0)),
                pl.BlockSpec((1, ci), lambda i: (0, 0)),
            ],
            out_specs=conv_out_specs(),
            compiler_params=_compiler_params(),
        )(yk, wall, ps, pss, gi.reshape(1, -1), bi.reshape(1, -1))

    out = pl.pallas_call(
        functools.partial(_epilogue_kernel, wp=wp, count=count),
        out_shape=jax.ShapeDtypeStruct((n, cout, h2, w2s), jnp.float32),
        grid=(n,),
        in_specs=[
            pl.BlockSpec((1, cout, lf), lambda i: (i, 0, 0)),
            pl.BlockSpec((n, 1, cout), lambda i: (0, 0, 0)),
            pl.BlockSpec((n, 1, cout), lambda i: (0, 0, 0)),
            pl.BlockSpec((1, cout), lambda i: (0, 0)),
            pl.BlockSpec((1, cout), lambda i: (0, 0)),
        ],
        out_specs=pl.BlockSpec((1, cout, h2, w2s), lambda i: (i, 0, 0, 0)),
        compiler_params=_compiler_params(),
        )(yk, ps, pss, gamma3.reshape(1, -1), beta3.reshape(1, -1))
    return out


# speedup vs baseline: 1.0568x; 1.0568x over previous
import functools

import jax
import jax.numpy as jnp
import numpy as np
from jax.experimental import pallas as pl
from jax.experimental.pallas import tpu as pltpu

_CD = jnp.bfloat16
_NT = 3
_EPS = 1e-5


def _compiler_params():
    return pltpu.CompilerParams(
        dimension_semantics=("parallel",),
        vmem_limit_bytes=56 * 1024 * 1024)


def _up_vec(n, m):
    k = np.arange(m) // 2
    p = np.arange(m) % 2
    nb = np.where(p == 0, np.maximum(k - 1, 0), np.minimum(k + 1, n - 1))
    r = np.arange(n)[:, None]
    return 0.75 * (r == k[None, :]) + 0.25 * (r == nb[None, :])


def _upsample_matrix(h, w, wp):
    uv = _up_vec(h, 2 * h)
    uh = np.pad(_up_vec(w, 2 * w), ((0, 0), (1, 1)))
    b = uv[:, None, :, None] * uh[None, :, None, :]
    return jnp.asarray(b.reshape(h * w, 2 * h * wp), dtype=_CD)


def _fold_bn_in_kernel(psin_ref, pssin_ref, gamma_ref, beta_ref, count):
    s = jnp.sum(psin_ref[...], axis=0)
    ss = jnp.sum(pssin_ref[...], axis=0)
    mean = s / count
    var = jnp.maximum(ss / count - mean * mean, 0.0)
    scale = gamma_ref[...] / jnp.sqrt(var + _EPS)
    shift = beta_ref[...] - mean * scale
    return scale, shift


def _stage_taps(a, stk9_ref, *, wp, xs):
    cin, lf = a.shape
    xfull = jnp.concatenate(
        [jnp.zeros((cin, wp + 1), _CD), a,
         jnp.zeros((cin, xs + 2 * wp + 2 - (wp + 1) - lf), _CD)], axis=1)
    for k in range(9):
        s = (k // 3) * wp + (k % 3)
        stk9_ref[k * cin:(k + 1) * cin, :] = xfull[:, s:s + xs]


def _conv_tiles(stk9_ref, w_ref, y_ref, ps_ref, pss_ref, *, wp, lt):
    ps = jnp.zeros((1, y_ref.shape[0]), jnp.float32)
    pss = jnp.zeros((1, y_ref.shape[0]), jnp.float32)
    for ti in range(_NT):
        f0 = ti * lt
        stk = stk9_ref[:, f0:f0 + lt]
        acc = jnp.dot(w_ref[...], stk, preferred_element_type=jnp.float32)
        y_ref[:, f0:f0 + lt] = acc.astype(y_ref.dtype)
        col = (jax.lax.broadcasted_iota(jnp.int32, (1, lt), 1) + f0) % wp
        am = jnp.where((col >= 1) & (col <= wp - 2), acc, 0.0)
        ps = ps + jnp.sum(am, axis=1, keepdims=True).reshape(1, -1)
        pss = pss + jnp.sum(am * am, axis=1, keepdims=True).reshape(1, -1)
    ps_ref[...] = ps
    pss_ref[...] = pss


def _conv1_kernel(x_ref, prev_ref, b_ref, w_ref, y_ref, ps_ref, pss_ref,
                  stk9_ref, *, wp, lt, xs):
    for b in range(x_ref.shape[0]):
        ax = jnp.dot(x_ref[b].astype(_CD), b_ref[...],
                     preferred_element_type=jnp.float32).astype(_CD)
        a = jnp.concatenate([ax, prev_ref[b]], axis=0)
        _stage_taps(a, stk9_ref, wp=wp, xs=xs)
        _conv_tiles(stk9_ref, w_ref, y_ref.at[b], ps_ref.at[b], pss_ref.at[b],
                    wp=wp, lt=lt)


def _conv_mid_kernel(yin_ref, w_ref, psin_ref, pssin_ref, gamma_ref, beta_ref,
                     y_ref, ps_ref, pss_ref, stk9_ref, *, wp, lt, xs, count):
    lf = yin_ref.shape[2]
    scale, shift = _fold_bn_in_kernel(psin_ref, pssin_ref, gamma_ref,
                                      beta_ref, count)
    col = jax.lax.broadcasted_iota(jnp.int32, (1, lf), 1) % wp
    for b in range(yin_ref.shape[0]):
        v = yin_ref[b].astype(jnp.float32) * scale.reshape(-1, 1) \
            + shift.reshape(-1, 1)
        v = jnp.maximum(v, 0.0)
        a = jnp.where((col >= 1) & (col <= wp - 2), v, 0.0).astype(_CD)
        _stage_taps(a, stk9_ref, wp=wp, xs=xs)
        _conv_tiles(stk9_ref, w_ref, y_ref.at[b], ps_ref.at[b], pss_ref.at[b],
                    wp=wp, lt=lt)


def _epilogue_kernel(yin_ref, psin_ref, pssin_ref, gamma_ref, beta_ref,
                     o_ref, *, wp, count):
    c, h2, w2 = o_ref.shape[1], o_ref.shape[2], o_ref.shape[3]
    scale, shift = _fold_bn_in_kernel(psin_ref, pssin_ref, gamma_ref,
                                      beta_ref, count)
    for b in range(yin_ref.shape[0]):
        v = jnp.maximum(yin_ref[b].astype(jnp.float32) * scale.reshape(-1, 1)
                        + shift.reshape(-1, 1), 0.0)
        o_ref[b] = v.reshape(c, h2, wp)[:, :, 1:w2 + 1]


def _stack_taps(w_oihw):
    cout, cin = w_oihw.shape[0], w_oihw.shape[1]
    return jnp.transpose(w_oihw, (0, 2, 3, 1)).reshape(cout, 9 * cin).astype(_CD)


def kernel(prev, x, w1, b1, gamma1, beta1, w2, b2, gamma2, beta2,
           w3, b3, gamma3, beta3):
    n, cx, h, w = x.shape
    cp = prev.shape[1]
    h2, w2s = 2 * h, 2 * w
    wp = w2s + 2
    lf = h2 * wp
    lt = lf // _NT
    count = n * h2 * w2s
    cin1 = cx + cp
    cout = w1.shape[0]
    xs = -(-lf // 128) * 128 + 256
    g = 2 if n % 2 == 0 else 1

    xf = x.reshape(n, cx, h * w)
    prevf = jnp.pad(prev, ((0, 0), (0, 0), (0, 0), (1, 1))
                    ).reshape(n, cp, lf).astype(_CD)
    bmat = _upsample_matrix(h, w, wp)
    wall = jnp.concatenate(
        [_stack_taps(w1), _stack_taps(w2), _stack_taps(w3)], axis=1)

    def conv_out_shape():
        return (jax.ShapeDtypeStruct((n, cout, lf), _CD),
                jax.ShapeDtypeStruct((n, 1, cout), jnp.float32),
                jax.ShapeDtypeStruct((n, 1, cout), jnp.float32))

    def conv_out_specs():
        return (pl.BlockSpec((g, cout, lf), lambda i: (i, 0, 0)),
                pl.BlockSpec((g, 1, cout), lambda i: (i, 0, 0)),
                pl.BlockSpec((g, 1, cout), lambda i: (i, 0, 0)))

    y1, ps, pss = pl.pallas_call(
        functools.partial(_conv1_kernel, wp=wp, lt=lt, xs=xs),
        out_shape=conv_out_shape(),
        grid=(n // g,),
        in_specs=[
            pl.BlockSpec((g, cx, h * w), lambda i: (i, 0, 0)),
            pl.BlockSpec((g, cp, lf), lambda i: (i, 0, 0)),
            pl.BlockSpec((h * w, lf), lambda i: (0, 0)),
            pl.BlockSpec((cout, 9 * cin1), lambda i: (0, 0)),
        ],
        out_specs=conv_out_specs(),
        scratch_shapes=[pltpu.VMEM((9 * cin1, xs), _CD)],
        compiler_params=_compiler_params(),
    )(xf, prevf, bmat, wall)

    yk = y1
    for li, (wi, gi, bi) in enumerate(((w2, gamma1, beta1),
                                       (w3, gamma2, beta2))):
        ci = wi.shape[1]
        wblk = (9 * cin1) // (9 * ci) + li
        yk, ps, pss = pl.pallas_call(
            functools.partial(_conv_mid_kernel, wp=wp, lt=lt, xs=xs,
                              count=count),
            out_shape=conv_out_shape(),
            grid=(n // g,),
            in_specs=[
                pl.BlockSpec((g, ci, lf), lambda i: (i, 0, 0)),
                pl.BlockSpec((cout, 9 * ci),
                             functools.partial(lambda b, i: (0, b), wblk)),
                pl.BlockSpec((n, 1, ci), lambda i: (0, 0, 0)),
                pl.BlockSpec((n, 1, ci), lambda i: (0, 0, 0)),
                pl.BlockSpec((1, ci), lambda i: (0, 0)),
                pl.BlockSpec((1, ci), lambda i: (0, 0)),
            ],
            out_specs=conv_out_specs(),
            scratch_shapes=[pltpu.VMEM((9 * ci, xs), _CD)],
            compiler_params=_compiler_params(),
        )(yk, wall, ps, pss, gi.reshape(1, -1), bi.reshape(1, -1))

    out = pl.pallas_call(
        functools.partial(_epilogue_kernel, wp=wp, count=count),
        out_shape=jax.ShapeDtypeStruct((n, cout, h2, w2s), jnp.float32),
        grid=(n // g,),
        in_specs=[
            pl.BlockSpec((g, cout, lf), lambda i: (i, 0, 0)),
            pl.BlockSpec((n, 1, cout), lambda i: (0, 0, 0)),
            pl.BlockSpec((n, 1, cout), lambda i: (0, 0, 0)),
            pl.BlockSpec((1, cout), lambda i: (0, 0)),
            pl.BlockSpec((1, cout), lambda i: (0, 0)),
        ],
        out_specs=pl.BlockSpec((g, cout, h2, w2s), lambda i: (i, 0, 0, 0)),
        compiler_params=_compiler_params(),
        )(yk, ps, pss, gamma3.reshape(1, -1), beta3.reshape(1, -1))
    return out


# speedup vs baseline: 1.1160x; 1.0560x over previous
import functools

import jax
import jax.numpy as jnp
import numpy as np
from jax.experimental import pallas as pl
from jax.experimental.pallas import tpu as pltpu

_CD = jnp.bfloat16
_NT = 1
_EPS = 1e-5


def _compiler_params():
    return pltpu.CompilerParams(
        dimension_semantics=("parallel",),
        vmem_limit_bytes=56 * 1024 * 1024)


def _up_vec(n, m):
    k = np.arange(m) // 2
    p = np.arange(m) % 2
    nb = np.where(p == 0, np.maximum(k - 1, 0), np.minimum(k + 1, n - 1))
    r = np.arange(n)[:, None]
    return 0.75 * (r == k[None, :]) + 0.25 * (r == nb[None, :])


def _upsample_matrix(h, w, wp):
    uv = _up_vec(h, 2 * h)
    uh = np.pad(_up_vec(w, 2 * w), ((0, 0), (1, 1)))
    b = uv[:, None, :, None] * uh[None, :, None, :]
    return jnp.asarray(b.reshape(h * w, 2 * h * wp), dtype=_CD)


def _fold_bn_in_kernel(psin_ref, pssin_ref, gamma_ref, beta_ref, count):
    s = jnp.sum(psin_ref[...], axis=0)
    ss = jnp.sum(pssin_ref[...], axis=0)
    mean = s / count
    var = jnp.maximum(ss / count - mean * mean, 0.0)
    scale = gamma_ref[...] / jnp.sqrt(var + _EPS)
    shift = beta_ref[...] - mean * scale
    return scale, shift


def _stage_taps(a, stk9_ref, *, wp, xs):
    cin, lf = a.shape
    xfull = jnp.concatenate(
        [jnp.zeros((cin, wp + 1), _CD), a,
         jnp.zeros((cin, xs + 2 * wp + 2 - (wp + 1) - lf), _CD)], axis=1)
    for k in range(9):
        s = (k // 3) * wp + (k % 3)
        stk9_ref[k * cin:(k + 1) * cin, :] = xfull[:, s:s + xs]


def _conv_tiles(stk9_ref, w_ref, y_ref, ps_ref, pss_ref, *, wp, lt):
    ps = jnp.zeros((1, y_ref.shape[1]), jnp.float32)
    pss = jnp.zeros((1, y_ref.shape[1]), jnp.float32)
    for ti in range(_NT):
        f0 = ti * lt
        stk = stk9_ref[:, f0:f0 + lt]
        acc = jnp.dot(w_ref[...], stk, preferred_element_type=jnp.float32)
        y_ref[0, :, f0:f0 + lt] = acc.astype(y_ref.dtype)
        col = (jax.lax.broadcasted_iota(jnp.int32, (1, lt), 1) + f0) % wp
        am = jnp.where((col >= 1) & (col <= wp - 2), acc, 0.0)
        ps = ps + jnp.sum(am, axis=1, keepdims=True).reshape(1, -1)
        pss = pss + jnp.sum(am * am, axis=1, keepdims=True).reshape(1, -1)
    ps_ref[0] = ps
    pss_ref[0] = pss


def _conv1_kernel(x_ref, prev_ref, b_ref, w_ref, y_ref, ps_ref, pss_ref,
                  stk9_ref, *, wp, lt, xs):
    ax = jnp.dot(x_ref[0].astype(_CD), b_ref[...],
                 preferred_element_type=jnp.float32).astype(_CD)
    a = jnp.concatenate([ax, prev_ref[0]], axis=0)
    _stage_taps(a, stk9_ref, wp=wp, xs=xs)
    _conv_tiles(stk9_ref, w_ref, y_ref, ps_ref, pss_ref, wp=wp, lt=lt)


def _conv_mid_kernel(yin_ref, w_ref, psin_ref, pssin_ref, gamma_ref, beta_ref,
                     y_ref, ps_ref, pss_ref, stk9_ref, *, wp, lt, xs, count):
    lf = yin_ref.shape[2]
    scale, shift = _fold_bn_in_kernel(psin_ref, pssin_ref, gamma_ref,
                                      beta_ref, count)
    v = yin_ref[0].astype(jnp.float32) * scale.reshape(-1, 1) + shift.reshape(-1, 1)
    v = jnp.maximum(v, 0.0)
    col = jax.lax.broadcasted_iota(jnp.int32, (1, lf), 1) % wp
    a = jnp.where((col >= 1) & (col <= wp - 2), v, 0.0).astype(_CD)
    _stage_taps(a, stk9_ref, wp=wp, xs=xs)
    _conv_tiles(stk9_ref, w_ref, y_ref, ps_ref, pss_ref, wp=wp, lt=lt)


def _epilogue_kernel(yin_ref, psin_ref, pssin_ref, gamma_ref, beta_ref,
                     o_ref, *, wp, count):
    c, h2, w2 = o_ref.shape[1], o_ref.shape[2], o_ref.shape[3]
    scale, shift = _fold_bn_in_kernel(psin_ref, pssin_ref, gamma_ref,
                                      beta_ref, count)
    v = jnp.maximum(yin_ref[0].astype(jnp.float32) * scale.reshape(-1, 1)
                    + shift.reshape(-1, 1), 0.0)
    o_ref[0] = v.reshape(c, h2, wp)[:, :, 1:w2 + 1]


def _stack_taps(w_oihw):
    cout, cin = w_oihw.shape[0], w_oihw.shape[1]
    return jnp.transpose(w_oihw, (0, 2, 3, 1)).reshape(cout, 9 * cin).astype(_CD)


def kernel(prev, x, w1, b1, gamma1, beta1, w2, b2, gamma2, beta2,
           w3, b3, gamma3, beta3):
    n, cx, h, w = x.shape
    cp = prev.shape[1]
    h2, w2s = 2 * h, 2 * w
    wp = w2s + 2
    lf = h2 * wp
    lt = lf // _NT
    count = n * h2 * w2s
    cin1 = cx + cp
    cout = w1.shape[0]
    xs = -(-lf // 128) * 128 + 256

    xf = x.reshape(n, cx, h * w)
    prevf = jnp.pad(prev, ((0, 0), (0, 0), (0, 0), (1, 1))
                    ).reshape(n, cp, lf).astype(_CD)
    bmat = _upsample_matrix(h, w, wp)
    wall = jnp.concatenate(
        [_stack_taps(w1), _stack_taps(w2), _stack_taps(w3)], axis=1)

    def conv_out_shape():
        return (jax.ShapeDtypeStruct((n, cout, lf), _CD),
                jax.ShapeDtypeStruct((n, 1, cout), jnp.float32),
                jax.ShapeDtypeStruct((n, 1, cout), jnp.float32))

    def conv_out_specs():
        return (pl.BlockSpec((1, cout, lf), lambda i: (i, 0, 0)),
                pl.BlockSpec((1, 1, cout), lambda i: (i, 0, 0)),
                pl.BlockSpec((1, 1, cout), lambda i: (i, 0, 0)))

    y1, ps, pss = pl.pallas_call(
        functools.partial(_conv1_kernel, wp=wp, lt=lt, xs=xs),
        out_shape=conv_out_shape(),
        grid=(n,),
        in_specs=[
            pl.BlockSpec((1, cx, h * w), lambda i: (i, 0, 0)),
            pl.BlockSpec((1, cp, lf), lambda i: (i, 0, 0)),
            pl.BlockSpec((h * w, lf), lambda i: (0, 0)),
            pl.BlockSpec((cout, 9 * cin1), lambda i: (0, 0)),
        ],
        out_specs=conv_out_specs(),
        scratch_shapes=[pltpu.VMEM((9 * cin1, xs), _CD)],
        compiler_params=_compiler_params(),
    )(xf, prevf, bmat, wall)

    yk = y1
    for li, (wi, gi, bi) in enumerate(((w2, gamma1, beta1),
                                       (w3, gamma2, beta2))):
        ci = wi.shape[1]
        wblk = (9 * cin1) // (9 * ci) + li
        yk, ps, pss = pl.pallas_call(
            functools.partial(_conv_mid_kernel, wp=wp, lt=lt, xs=xs,
                              count=count),
            out_shape=conv_out_shape(),
            grid=(n,),
            in_specs=[
                pl.BlockSpec((1, ci, lf), lambda i: (i, 0, 0)),
                pl.BlockSpec((cout, 9 * ci),
                             functools.partial(lambda b, i: (0, b), wblk)),
                pl.BlockSpec((n, 1, ci), lambda i: (0, 0, 0)),
                pl.BlockSpec((n, 1, ci), lambda i: (0, 0, 0)),
                pl.BlockSpec((1, ci), lambda i: (0, 0)),
                pl.BlockSpec((1, ci), lambda i: (0, 0)),
            ],
            out_specs=conv_out_specs(),
            scratch_shapes=[pltpu.VMEM((9 * ci, xs), _CD)],
            compiler_params=_compiler_params(),
        )(yk, wall, ps, pss, gi.reshape(1, -1), bi.reshape(1, -1))

    out = pl.pallas_call(
        functools.partial(_epilogue_kernel, wp=wp, count=count),
        out_shape=jax.ShapeDtypeStruct((n, cout, h2, w2s), jnp.float32),
        grid=(n,),
        in_specs=[
            pl.BlockSpec((1, cout, lf), lambda i: (i, 0, 0)),
            pl.BlockSpec((n, 1, cout), lambda i: (0, 0, 0)),
            pl.BlockSpec((n, 1, cout), lambda i: (0, 0, 0)),
            pl.BlockSpec((1, cout), lambda i: (0, 0)),
            pl.BlockSpec((1, cout), lambda i: (0, 0)),
        ],
        out_specs=pl.BlockSpec((1, cout, h2, w2s), lambda i: (i, 0, 0, 0)),
        compiler_params=_compiler_params(),
        )(yk, ps, pss, gamma3.reshape(1, -1), beta3.reshape(1, -1))
    return out


# speedup vs baseline: 1.1197x; 1.0034x over previous
import functools

import jax
import jax.numpy as jnp
import numpy as np
from jax.experimental import pallas as pl
from jax.experimental.pallas import tpu as pltpu

_CD = jnp.bfloat16
_NT = 3
_EPS = 1e-5


def _compiler_params():
    return pltpu.CompilerParams(
        dimension_semantics=("parallel",),
        vmem_limit_bytes=56 * 1024 * 1024)


def _up_vec(n, m):
    k = np.arange(m) // 2
    p = np.arange(m) % 2
    nb = np.where(p == 0, np.maximum(k - 1, 0), np.minimum(k + 1, n - 1))
    r = np.arange(n)[:, None]
    return 0.75 * (r == k[None, :]) + 0.25 * (r == nb[None, :])


def _upsample_matrix(h, w, wp):
    uv = _up_vec(h, 2 * h)
    uh = np.pad(_up_vec(w, 2 * w), ((0, 0), (1, 1)))
    b = uv[:, None, :, None] * uh[None, :, None, :]
    return jnp.asarray(b.reshape(h * w, 2 * h * wp), dtype=_CD)


def _fold_bn_in_kernel(psin_ref, pssin_ref, gamma_ref, beta_ref, count):
    s = jnp.sum(psin_ref[...], axis=0)
    ss = jnp.sum(pssin_ref[...], axis=0)
    mean = s / count
    var = jnp.maximum(ss / count - mean * mean, 0.0)
    scale = gamma_ref[...] / jnp.sqrt(var + _EPS)
    shift = beta_ref[...] - mean * scale
    return scale, shift


def _stage_taps(a, stk9_ref, *, wp, xs):
    cin, lf = a.shape
    xfull = jnp.concatenate(
        [jnp.zeros((cin, wp + 1), _CD), a,
         jnp.zeros((cin, xs + 2 * wp + 2 - (wp + 1) - lf), _CD)], axis=1)
    for k in range(9):
        s = (k // 3) * wp + (k % 3)
        stk9_ref[k * cin:(k + 1) * cin, :] = xfull[:, s:s + xs]


def _conv_tiles(stk9_ref, w_ref, y_ref, ps_ref, pss_ref, *, wp, lt):
    ps = jnp.zeros((1, y_ref.shape[1]), jnp.float32)
    pss = jnp.zeros((1, y_ref.shape[1]), jnp.float32)
    for ti in range(_NT):
        f0 = ti * lt
        stk = stk9_ref[:, f0:f0 + lt]
        acc = jnp.dot(w_ref[...], stk, preferred_element_type=jnp.float32)
        y_ref[0, :, f0:f0 + lt] = acc.astype(y_ref.dtype)
        col = (jax.lax.broadcasted_iota(jnp.int32, (1, lt), 1) + f0) % wp
        am = jnp.where((col >= 1) & (col <= wp - 2), acc, 0.0)
        ps = ps + jnp.sum(am, axis=1, keepdims=True).reshape(1, -1)
        pss = pss + jnp.sum(am * am, axis=1, keepdims=True).reshape(1, -1)
    ps_ref[0] = ps
    pss_ref[0] = pss


def _conv1_kernel(x_ref, prev_ref, b_ref, w_ref, y_ref, ps_ref, pss_ref,
                  stk9_ref, *, wp, lt, xs):
    ax = jnp.dot(x_ref[0].astype(_CD), b_ref[...],
                 preferred_element_type=jnp.float32).astype(_CD)
    a = jnp.concatenate([ax, prev_ref[0]], axis=0)
    _stage_taps(a, stk9_ref, wp=wp, xs=xs)
    _conv_tiles(stk9_ref, w_ref, y_ref, ps_ref, pss_ref, wp=wp, lt=lt)


def _conv_mid_kernel(yin_ref, w_ref, psin_ref, pssin_ref, gamma_ref, beta_ref,
                     y_ref, ps_ref, pss_ref, stk9_ref, *, wp, lt, xs, count):
    lf = yin_ref.shape[2]
    scale, shift = _fold_bn_in_kernel(psin_ref, pssin_ref, gamma_ref,
                                      beta_ref, count)
    v = yin_ref[0].astype(jnp.float32) * scale.reshape(-1, 1) + shift.reshape(-1, 1)
    v = jnp.maximum(v, 0.0)
    col = jax.lax.broadcasted_iota(jnp.int32, (1, lf), 1) % wp
    a = jnp.where((col >= 1) & (col <= wp - 2), v, 0.0).astype(_CD)
    _stage_taps(a, stk9_ref, wp=wp, xs=xs)
    _conv_tiles(stk9_ref, w_ref, y_ref, ps_ref, pss_ref, wp=wp, lt=lt)


def _epilogue_kernel(yin_ref, psin_ref, pssin_ref, gamma_ref, beta_ref,
                     o_ref, *, wp, count):
    c, h2, w2 = o_ref.shape[1], o_ref.shape[2], o_ref.shape[3]
    scale, shift = _fold_bn_in_kernel(psin_ref, pssin_ref, gamma_ref,
                                      beta_ref, count)
    v = jnp.maximum(yin_ref[0].astype(jnp.float32) * scale.reshape(-1, 1)
                    + shift.reshape(-1, 1), 0.0)
    o_ref[0] = v.reshape(c, h2, wp)[:, :, 1:w2 + 1]


def _stack_taps(w_oihw):
    cout, cin = w_oihw.shape[0], w_oihw.shape[1]
    return jnp.transpose(w_oihw, (0, 2, 3, 1)).reshape(cout, 9 * cin).astype(_CD)


def kernel(prev, x, w1, b1, gamma1, beta1, w2, b2, gamma2, beta2,
           w3, b3, gamma3, beta3):
    n, cx, h, w = x.shape
    cp = prev.shape[1]
    h2, w2s = 2 * h, 2 * w
    wp = w2s + 2
    lf = h2 * wp
    lt = lf // _NT
    count = n * h2 * w2s
    cin1 = cx + cp
    cout = w1.shape[0]
    xs = lf

    xf = x.reshape(n, cx, h * w)
    prevf = jnp.pad(prev, ((0, 0), (0, 0), (0, 0), (1, 1))
                    ).reshape(n, cp, lf).astype(_CD)
    bmat = _upsample_matrix(h, w, wp)
    wall = jnp.concatenate(
        [_stack_taps(w1), _stack_taps(w2), _stack_taps(w3)], axis=1)

    def conv_out_shape():
        return (jax.ShapeDtypeStruct((n, cout, lf), _CD),
                jax.ShapeDtypeStruct((n, 1, cout), jnp.float32),
                jax.ShapeDtypeStruct((n, 1, cout), jnp.float32))

    def conv_out_specs():
        return (pl.BlockSpec((1, cout, lf), lambda i: (i, 0, 0)),
                pl.BlockSpec((1, 1, cout), lambda i: (i, 0, 0)),
                pl.BlockSpec((1, 1, cout), lambda i: (i, 0, 0)))

    y1, ps, pss = pl.pallas_call(
        functools.partial(_conv1_kernel, wp=wp, lt=lt, xs=xs),
        out_shape=conv_out_shape(),
        grid=(n,),
        in_specs=[
            pl.BlockSpec((1, cx, h * w), lambda i: (i, 0, 0)),
            pl.BlockSpec((1, cp, lf), lambda i: (i, 0, 0)),
            pl.BlockSpec((h * w, lf), lambda i: (0, 0)),
            pl.BlockSpec((cout, 9 * cin1), lambda i: (0, 0)),
        ],
        out_specs=conv_out_specs(),
        scratch_shapes=[pltpu.VMEM((9 * cin1, xs), _CD)],
        compiler_params=_compiler_params(),
    )(xf, prevf, bmat, wall)

    yk = y1
    for li, (wi, gi, bi) in enumerate(((w2, gamma1, beta1),
                                       (w3, gamma2, beta2))):
        ci = wi.shape[1]
        wblk = (9 * cin1) // (9 * ci) + li
        yk, ps, pss = pl.pallas_call(
            functools.partial(_conv_mid_kernel, wp=wp, lt=lt, xs=xs,
                              count=count),
            out_shape=conv_out_shape(),
            grid=(n,),
            in_specs=[
                pl.BlockSpec((1, ci, lf), lambda i: (i, 0, 0)),
                pl.BlockSpec((cout, 9 * ci),
                             functools.partial(lambda b, i: (0, b), wblk)),
                pl.BlockSpec((n, 1, ci), lambda i: (0, 0, 0)),
                pl.BlockSpec((n, 1, ci), lambda i: (0, 0, 0)),
                pl.BlockSpec((1, ci), lambda i: (0, 0)),
                pl.BlockSpec((1, ci), lambda i: (0, 0)),
            ],
            out_specs=conv_out_specs(),
            scratch_shapes=[pltpu.VMEM((9 * ci, xs), _CD)],
            compiler_params=_compiler_params(),
        )(yk, wall, ps, pss, gi.reshape(1, -1), bi.reshape(1, -1))

    out = pl.pallas_call(
        functools.partial(_epilogue_kernel, wp=wp, count=count),
        out_shape=jax.ShapeDtypeStruct((n, cout, h2, w2s), jnp.float32),
        grid=(n,),
        in_specs=[
            pl.BlockSpec((1, cout, lf), lambda i: (i, 0, 0)),
            pl.BlockSpec((n, 1, cout), lambda i: (0, 0, 0)),
            pl.BlockSpec((n, 1, cout), lambda i: (0, 0, 0)),
            pl.BlockSpec((1, cout), lambda i: (0, 0)),
            pl.BlockSpec((1, cout), lambda i: (0, 0)),
        ],
        out_specs=pl.BlockSpec((1, cout, h2, w2s), lambda i: (i, 0, 0, 0)),
        compiler_params=_compiler_params(),
        )(yk, ps, pss, gamma3.reshape(1, -1), beta3.reshape(1, -1))
    return out
